# Initial kernel scaffold; baseline (speedup 1.0000x reference)
import jax
import jax.numpy as jnp
from jax.experimental import pallas as pl


def kernel(x, edge_index, batch, W0, b0, W1, b1, W2, b2, Wc, bc):
    raise NotImplementedError("write your pallas kernel here")


# speedup vs baseline: 7.4336x; 7.4336x over previous
import functools

import jax
import jax.numpy as jnp
from jax import lax
from jax.experimental import pallas as pl
from jax.experimental.pallas import tpu as pltpu
from jax.experimental.pallas import tpu_sc as plsc

N = 10000
E = 160000
DIM = 256
QW = 64
NQ = 4
NUM_CLS = 64
NUM_GRAPHS = 64

N_ACC = 10240
ROWS_PER_TILE = N_ACC // 16
CHUNK = 128
CHUNKS_PER_TILE = 80
E_PAD = 16 * CHUNKS_PER_TILE * CHUNK
NBUF = 4
GROUPS = CHUNKS_PER_TILE // NBUF

_sc_mesh = plsc.VectorSubcoreMesh(core_axis_name="c", subcore_axis_name="s")
_sc_params = pltpu.CompilerParams(use_tc_tiling_on_sc=False)


def _deg_body(gdst_hbm, ones_hbm, zeros_hbm, deg_out, didx, ones_v, zbuf,
              acc_sh):
    cid = lax.axis_index("c")
    sid = lax.axis_index("s")

    @pl.when(cid == 0)
    def _():
        pltpu.sync_copy(zeros_hbm, zbuf)
        pltpu.sync_copy(zbuf, acc_sh.at[pl.ds(sid * ROWS_PER_TILE,
                                              ROWS_PER_TILE)])
        pltpu.sync_copy(ones_hbm, ones_v)
        pltpu.sync_copy(gdst_hbm.at[sid], didx)

    plsc.subcore_barrier()

    @pl.when(cid == 0)
    def _():
        def body(j, carry):
            pltpu.sync_copy(ones_v, acc_sh.at[didx.at[j]], add=True)
            return carry
        lax.fori_loop(0, CHUNKS_PER_TILE, body, 0)

    plsc.subcore_barrier()

    @pl.when(cid == 0)
    def _():
        pltpu.sync_copy(acc_sh.at[pl.ds(sid * ROWS_PER_TILE, ROWS_PER_TILE)],
                        zbuf)
        pltpu.sync_copy(zbuf, deg_out.at[pl.ds(sid * ROWS_PER_TILE,
                                               ROWS_PER_TILE)])


_deg_kernel = functools.partial(
    pl.kernel,
    out_type=jax.ShapeDtypeStruct((N_ACC, 16), jnp.float32),
    mesh=_sc_mesh,
    scratch_types=[
        pltpu.VMEM((CHUNKS_PER_TILE, CHUNK), jnp.int32),
        pltpu.VMEM((CHUNK, 16), jnp.float32),
        pltpu.VMEM((ROWS_PER_TILE, 16), jnp.float32),
        pltpu.VMEM_SHARED((N_ACC, 16), jnp.float32),
    ],
    compiler_params=_sc_params,
)(_deg_body)


def _agg_body(p_hbm, gsrc_hbm, gdst_hbm, out_hbm, sidx, didx, b0, b1, b2, b3,
              s0, s1, s2, s3, acc_sh):
    cid = lax.axis_index("c")
    sid = lax.axis_index("s")
    bufs = [b0, b1, b2, b3]
    sems = [s0, s1, s2, s3]
    rowbase = sid * ROWS_PER_TILE

    pltpu.sync_copy(gdst_hbm.at[sid], didx)

    for qi in range(2):
        q = 2 * cid + qi
        hbase = q * N_ACC + rowbase
        for k in range(ROWS_PER_TILE // CHUNK):
            pltpu.sync_copy(p_hbm.at[pl.ds(hbase + k * CHUNK, CHUNK)], b0)
            pltpu.sync_copy(b0, acc_sh.at[pl.ds(rowbase + k * CHUNK, CHUNK)])
        pltpu.sync_copy(gsrc_hbm.at[q, sid], sidx)

        plsc.subcore_barrier()

        def group(g, carry):
            j0 = g * NBUF
            descs = []
            for k in range(NBUF):
                descs.append(
                    pltpu.async_copy(p_hbm.at[sidx.at[j0 + k]], bufs[k],
                                     sems[k]))
            for k in range(NBUF):
                descs[k].wait()
                pltpu.sync_copy(bufs[k], acc_sh.at[didx.at[j0 + k]], add=True)
            return carry

        lax.fori_loop(0, GROUPS, group, 0)

        plsc.subcore_barrier()

        for k in range(ROWS_PER_TILE // CHUNK):
            pltpu.sync_copy(acc_sh.at[pl.ds(rowbase + k * CHUNK, CHUNK)], b0)
            pltpu.sync_copy(b0, out_hbm.at[pl.ds(hbase + k * CHUNK, CHUNK)])


_agg_kernel = functools.partial(
    pl.kernel,
    out_type=jax.ShapeDtypeStruct((NQ * N_ACC, QW), jnp.float32),
    mesh=_sc_mesh,
    scratch_types=[
        pltpu.VMEM((CHUNKS_PER_TILE, CHUNK), jnp.int32),
        pltpu.VMEM((CHUNKS_PER_TILE, CHUNK), jnp.int32),
    ] + [pltpu.VMEM((CHUNK, QW), jnp.float32)] * NBUF
      + [pltpu.SemaphoreType.DMA] * NBUF
      + [pltpu.VMEM_SHARED((N_ACC, QW), jnp.float32)],
    compiler_params=_sc_params,
)(_agg_body)


_BN = 1280


def _tca_body(x_ref, w0_ref, b0_ref, w1_ref, deg_ref, out_ref):
    i = pl.program_id(0)
    h = jnp.dot(x_ref[...], w0_ref[...],
                preferred_element_type=jnp.float32) + b0_ref[...]
    t = jnp.dot(h, w1_ref[...], preferred_element_type=jnp.float32)
    dinv = lax.rsqrt(deg_ref[...] + 1.0)
    p = t * dinv
    rows = lax.broadcasted_iota(jnp.int32, (_BN, 1), 0) + i * _BN
    p = jnp.where(rows < N, p, 0.0)
    for q in range(NQ):
        out_ref[q] = p[:, q * QW:(q + 1) * QW]


def _tcb_body(acc_ref, deg_ref, b1_ref, w2_ref, out_ref):
    i = pl.program_id(0)
    a = jnp.concatenate([acc_ref[q] for q in range(NQ)], axis=1)
    dinv = lax.rsqrt(deg_ref[...] + 1.0)
    h1 = jnp.maximum(a * dinv + b1_ref[...], 0.0)
    t = jnp.dot(h1, w2_ref[...], preferred_element_type=jnp.float32)
    p = t * dinv
    rows = lax.broadcasted_iota(jnp.int32, (_BN, 1), 0) + i * _BN
    p = jnp.where(rows < N, p, 0.0)
    for q in range(NQ):
        out_ref[q] = p[:, q * QW:(q + 1) * QW]


_BNC = 1000


def _tcc_body(acc_ref, deg_ref, b2_ref, batch_ref, wc_ref, bc_ref, out_ref,
              sums_ref, cnts_ref):
    i = pl.program_id(0)

    @pl.when(i == 0)
    def _():
        sums_ref[...] = jnp.zeros_like(sums_ref)
        cnts_ref[...] = jnp.zeros_like(cnts_ref)

    a = jnp.concatenate([acc_ref[q] for q in range(NQ)], axis=1)
    dinv = lax.rsqrt(deg_ref[...] + 1.0)
    h2 = a * dinv + b2_ref[...]
    b = batch_ref[0, 0]
    gids = lax.broadcasted_iota(jnp.int32, (NUM_GRAPHS, _BNC), 0)
    bt = lax.broadcast_in_dim(b, (NUM_GRAPHS, _BNC), (1,))
    onehot = (bt == gids).astype(jnp.float32)
    sums_ref[...] += lax.dot_general(
        onehot, h2, (((1,), (0,)), ((), ())),
        preferred_element_type=jnp.float32)
    cnts_ref[...] += jnp.sum(onehot, axis=1, keepdims=True) * jnp.ones(
        (NUM_GRAPHS, 128), jnp.float32)

    @pl.when(i == N // _BNC - 1)
    def _():
        pooled = sums_ref[...] / jnp.maximum(cnts_ref[:, 0:1], 1.0)
        logits = jnp.dot(pooled, wc_ref[...],
                         preferred_element_type=jnp.float32) + bc_ref[...]
        m = jnp.max(logits, axis=1, keepdims=True)
        e = jnp.exp(logits - m)
        out_ref[...] = e / jnp.sum(e, axis=1, keepdims=True)


def _tca(x_pad, W0, b0, W1, deg_col):
    return pl.pallas_call(
        _tca_body,
        grid=(N_ACC // _BN,),
        in_specs=[
            pl.BlockSpec((_BN, DIM), lambda i: (i, 0)),
            pl.BlockSpec((DIM, DIM), lambda i: (0, 0)),
            pl.BlockSpec((1, DIM), lambda i: (0, 0)),
            pl.BlockSpec((DIM, DIM), lambda i: (0, 0)),
            pl.BlockSpec((_BN, 1), lambda i: (i, 0)),
        ],
        out_specs=pl.BlockSpec((NQ, _BN, QW), lambda i: (0, i, 0)),
        out_shape=jax.ShapeDtypeStruct((NQ, N_ACC, QW), jnp.float32),
    )(x_pad, W0, b0.reshape(1, DIM), W1, deg_col)


def _tcb(acc, deg_col, b1, W2):
    return pl.pallas_call(
        _tcb_body,
        grid=(N_ACC // _BN,),
        in_specs=[
            pl.BlockSpec((NQ, _BN, QW), lambda i: (0, i, 0)),
            pl.BlockSpec((_BN, 1), lambda i: (i, 0)),
            pl.BlockSpec((1, DIM), lambda i: (0, 0)),
            pl.BlockSpec((DIM, DIM), lambda i: (0, 0)),
        ],
        out_specs=pl.BlockSpec((NQ, _BN, QW), lambda i: (0, i, 0)),
        out_shape=jax.ShapeDtypeStruct((NQ, N_ACC, QW), jnp.float32),
    )(acc, deg_col, b1.reshape(1, DIM), W2)


def _tcc(acc, deg_col, b2, batch3, Wc, bc):
    return pl.pallas_call(
        _tcc_body,
        grid=(N // _BNC,),
        in_specs=[
            pl.BlockSpec((NQ, _BNC, QW), lambda i: (0, i, 0)),
            pl.BlockSpec((_BNC, 1), lambda i: (i, 0)),
            pl.BlockSpec((1, DIM), lambda i: (0, 0)),
            pl.BlockSpec((1, 1, _BNC), lambda i: (i, 0, 0)),
            pl.BlockSpec((DIM, NUM_CLS), lambda i: (0, 0)),
            pl.BlockSpec((1, NUM_CLS), lambda i: (0, 0)),
        ],
        out_specs=pl.BlockSpec((NUM_GRAPHS, NUM_CLS), lambda i: (0, 0)),
        out_shape=jax.ShapeDtypeStruct((NUM_GRAPHS, NUM_CLS), jnp.float32),
        scratch_shapes=[
            pltpu.VMEM((NUM_GRAPHS, DIM), jnp.float32),
            pltpu.VMEM((NUM_GRAPHS, 128), jnp.float32),
        ],
    )(acc, deg_col, b2.reshape(1, DIM), batch3, Wc, bc.reshape(1, NUM_CLS))


def kernel(x, edge_index, batch, W0, b0, W1, b1, W2, b2, Wc, bc):
    src = edge_index[0]
    dst = edge_index[1]
    pad = E_PAD - E
    srcp = jnp.concatenate([src, jnp.full((pad,), N, jnp.int32)])
    dstp = jnp.concatenate([dst, jnp.full((pad,), N, jnp.int32)])
    rs = srcp.reshape(16, CHUNKS_PER_TILE, CHUNK)
    gsrc = jnp.stack([rs + q * N_ACC for q in range(NQ)])
    gdst = dstp.reshape(16, CHUNKS_PER_TILE, CHUNK)
    x_pad = jnp.pad(x, ((0, N_ACC - N), (0, 0)))
    batch3 = batch.reshape(N // _BNC, 1, _BNC)
    ones_in = jnp.ones((CHUNK, 16), jnp.float32)
    zeros_in = jnp.zeros((ROWS_PER_TILE, 16), jnp.float32)

    deg16 = _deg_kernel(gdst, ones_in, zeros_in)
    deg_col = deg16[:, 0:1]

    p1 = _tca(x_pad, W0, b0, W1, deg_col)
    acc1 = _agg_kernel(p1.reshape(NQ * N_ACC, QW), gsrc, gdst)
    p2 = _tcb(acc1.reshape(NQ, N_ACC, QW), deg_col, b1, W2)
    acc2 = _agg_kernel(p2.reshape(NQ * N_ACC, QW), gsrc, gdst)
    return _tcc(acc2.reshape(NQ, N_ACC, QW), deg_col, b2, batch3, Wc, bc)


# speedup vs baseline: 7.9557x; 1.0702x over previous
import functools

import jax
import jax.numpy as jnp
from jax import lax
from jax.experimental import pallas as pl
from jax.experimental.pallas import tpu as pltpu
from jax.experimental.pallas import tpu_sc as plsc

N = 10000
E = 160000
DIM = 256
QW = 64
NQ = 4
NUM_CLS = 64
NUM_GRAPHS = 64

N_ACC = 10240
ROWS_PER_TILE = N_ACC // 16
CHUNK = 128
CHUNKS_PER_TILE = 80
E_PAD = 16 * CHUNKS_PER_TILE * CHUNK
NBUF = 8
GROUPS = CHUNKS_PER_TILE // NBUF

_sc_mesh = plsc.VectorSubcoreMesh(core_axis_name="c", subcore_axis_name="s")
_sc_params = pltpu.CompilerParams(use_tc_tiling_on_sc=False)


def _deg_body(gdst_hbm, ones_hbm, zeros_hbm, deg_out, didx, ones_v, zbuf,
              acc_sh):
    cid = lax.axis_index("c")
    sid = lax.axis_index("s")

    @pl.when(cid == 0)
    def _():
        pltpu.sync_copy(zeros_hbm, zbuf)
        pltpu.sync_copy(zbuf, acc_sh.at[pl.ds(sid * ROWS_PER_TILE,
                                              ROWS_PER_TILE)])
        pltpu.sync_copy(ones_hbm, ones_v)
        pltpu.sync_copy(gdst_hbm.at[sid], didx)

    plsc.subcore_barrier()

    @pl.when(cid == 0)
    def _():
        def body(j, carry):
            pltpu.sync_copy(ones_v, acc_sh.at[didx.at[j]], add=True)
            return carry
        lax.fori_loop(0, CHUNKS_PER_TILE, body, 0)

    plsc.subcore_barrier()

    @pl.when(cid == 0)
    def _():
        pltpu.sync_copy(acc_sh.at[pl.ds(sid * ROWS_PER_TILE, ROWS_PER_TILE)],
                        zbuf)
        pltpu.sync_copy(zbuf, deg_out.at[pl.ds(sid * ROWS_PER_TILE,
                                               ROWS_PER_TILE)])


_deg_kernel = functools.partial(
    pl.kernel,
    out_type=jax.ShapeDtypeStruct((N_ACC, 16), jnp.float32),
    mesh=_sc_mesh,
    scratch_types=[
        pltpu.VMEM((CHUNKS_PER_TILE, CHUNK), jnp.int32),
        pltpu.VMEM((CHUNK, 16), jnp.float32),
        pltpu.VMEM((ROWS_PER_TILE, 16), jnp.float32),
        pltpu.VMEM_SHARED((N_ACC, 16), jnp.float32),
    ],
    compiler_params=_sc_params,
)(_deg_body)


def _agg_body(p_hbm, gsrc_hbm, gdst_hbm, out_hbm, sidx, didx,
              b0, b1, b2, b3, b4, b5, b6, b7,
              s0, s1, s2, s3, s4, s5, s6, s7,
              t0, t1, t2, t3, t4, t5, t6, t7, acc_sh):
    cid = lax.axis_index("c")
    sid = lax.axis_index("s")
    bufs = [b0, b1, b2, b3, b4, b5, b6, b7]
    sems = [s0, s1, s2, s3, s4, s5, s6, s7]
    ssems = [t0, t1, t2, t3, t4, t5, t6, t7]
    rowbase = sid * ROWS_PER_TILE

    pltpu.sync_copy(gdst_hbm.at[sid], didx)

    for qi in range(2):
        q = 2 * cid + qi
        hbase = q * N_ACC + rowbase
        for k in range(ROWS_PER_TILE // CHUNK):
            pltpu.sync_copy(p_hbm.at[pl.ds(hbase + k * CHUNK, CHUNK)], b0)
            pltpu.sync_copy(b0, acc_sh.at[pl.ds(rowbase + k * CHUNK, CHUNK)])
        pltpu.sync_copy(gsrc_hbm.at[q, sid], sidx)

        plsc.subcore_barrier()

        def group(g, carry):
            j0 = g * NBUF
            gdescs = []
            for k in range(NBUF):
                gdescs.append(
                    pltpu.async_copy(p_hbm.at[sidx.at[j0 + k]], bufs[k],
                                     sems[k]))
            sdescs = []
            for k in range(NBUF):
                gdescs[k].wait()
                sdescs.append(
                    pltpu.async_copy(bufs[k], acc_sh.at[didx.at[j0 + k]],
                                     ssems[k], add=True))
            for k in range(NBUF):
                sdescs[k].wait()
            return carry

        lax.fori_loop(0, GROUPS, group, 0)

        plsc.subcore_barrier()

        for k in range(ROWS_PER_TILE // CHUNK):
            pltpu.sync_copy(acc_sh.at[pl.ds(rowbase + k * CHUNK, CHUNK)], b0)
            pltpu.sync_copy(b0, out_hbm.at[pl.ds(hbase + k * CHUNK, CHUNK)])


_agg_kernel = functools.partial(
    pl.kernel,
    out_type=jax.ShapeDtypeStruct((NQ * N_ACC, QW), jnp.float32),
    mesh=_sc_mesh,
    scratch_types=[
        pltpu.VMEM((CHUNKS_PER_TILE, CHUNK), jnp.int32),
        pltpu.VMEM((CHUNKS_PER_TILE, CHUNK), jnp.int32),
    ] + [pltpu.VMEM((CHUNK, QW), jnp.float32)] * NBUF
      + [pltpu.SemaphoreType.DMA] * (2 * NBUF)
      + [pltpu.VMEM_SHARED((N_ACC, QW), jnp.float32)],
    compiler_params=_sc_params,
)(_agg_body)


_BN = 1280


def _tca_body(x_ref, w0_ref, b0_ref, w1_ref, deg_ref, out_ref):
    i = pl.program_id(0)
    h = jnp.dot(x_ref[...], w0_ref[...],
                preferred_element_type=jnp.float32) + b0_ref[...]
    t = jnp.dot(h, w1_ref[...], preferred_element_type=jnp.float32)
    dinv = lax.rsqrt(deg_ref[...] + 1.0)
    p = t * dinv
    rows = lax.broadcasted_iota(jnp.int32, (_BN, 1), 0) + i * _BN
    p = jnp.where(rows < N, p, 0.0)
    for q in range(NQ):
        out_ref[q] = p[:, q * QW:(q + 1) * QW]


def _tcb_body(acc_ref, deg_ref, b1_ref, w2_ref, out_ref):
    i = pl.program_id(0)
    a = jnp.concatenate([acc_ref[q] for q in range(NQ)], axis=1)
    dinv = lax.rsqrt(deg_ref[...] + 1.0)
    h1 = jnp.maximum(a * dinv + b1_ref[...], 0.0)
    t = jnp.dot(h1, w2_ref[...], preferred_element_type=jnp.float32)
    p = t * dinv
    rows = lax.broadcasted_iota(jnp.int32, (_BN, 1), 0) + i * _BN
    p = jnp.where(rows < N, p, 0.0)
    for q in range(NQ):
        out_ref[q] = p[:, q * QW:(q + 1) * QW]


_BNC = 1000


def _tcc_body(acc_ref, deg_ref, b2_ref, batch_ref, wc_ref, bc_ref, out_ref,
              sums_ref, cnts_ref):
    i = pl.program_id(0)

    @pl.when(i == 0)
    def _():
        sums_ref[...] = jnp.zeros_like(sums_ref)
        cnts_ref[...] = jnp.zeros_like(cnts_ref)

    a = jnp.concatenate([acc_ref[q] for q in range(NQ)], axis=1)
    dinv = lax.rsqrt(deg_ref[...] + 1.0)
    h2 = a * dinv + b2_ref[...]
    b = batch_ref[0, 0]
    gids = lax.broadcasted_iota(jnp.int32, (NUM_GRAPHS, _BNC), 0)
    bt = lax.broadcast_in_dim(b, (NUM_GRAPHS, _BNC), (1,))
    onehot = (bt == gids).astype(jnp.float32)
    sums_ref[...] += lax.dot_general(
        onehot, h2, (((1,), (0,)), ((), ())),
        preferred_element_type=jnp.float32)
    cnts_ref[...] += jnp.sum(onehot, axis=1, keepdims=True) * jnp.ones(
        (NUM_GRAPHS, 128), jnp.float32)

    @pl.when(i == N // _BNC - 1)
    def _():
        pooled = sums_ref[...] / jnp.maximum(cnts_ref[:, 0:1], 1.0)
        logits = jnp.dot(pooled, wc_ref[...],
                         preferred_element_type=jnp.float32) + bc_ref[...]
        m = jnp.max(logits, axis=1, keepdims=True)
        e = jnp.exp(logits - m)
        out_ref[...] = e / jnp.sum(e, axis=1, keepdims=True)


def _tca(x_pad, W0, b0, W1, deg_col):
    return pl.pallas_call(
        _tca_body,
        grid=(N_ACC // _BN,),
        in_specs=[
            pl.BlockSpec((_BN, DIM), lambda i: (i, 0)),
            pl.BlockSpec((DIM, DIM), lambda i: (0, 0)),
            pl.BlockSpec((1, DIM), lambda i: (0, 0)),
            pl.BlockSpec((DIM, DIM), lambda i: (0, 0)),
            pl.BlockSpec((_BN, 1), lambda i: (i, 0)),
        ],
        out_specs=pl.BlockSpec((NQ, _BN, QW), lambda i: (0, i, 0)),
        out_shape=jax.ShapeDtypeStruct((NQ, N_ACC, QW), jnp.float32),
    )(x_pad, W0, b0.reshape(1, DIM), W1, deg_col)


def _tcb(acc, deg_col, b1, W2):
    return pl.pallas_call(
        _tcb_body,
        grid=(N_ACC // _BN,),
        in_specs=[
            pl.BlockSpec((NQ, _BN, QW), lambda i: (0, i, 0)),
            pl.BlockSpec((_BN, 1), lambda i: (i, 0)),
            pl.BlockSpec((1, DIM), lambda i: (0, 0)),
            pl.BlockSpec((DIM, DIM), lambda i: (0, 0)),
        ],
        out_specs=pl.BlockSpec((NQ, _BN, QW), lambda i: (0, i, 0)),
        out_shape=jax.ShapeDtypeStruct((NQ, N_ACC, QW), jnp.float32),
    )(acc, deg_col, b1.reshape(1, DIM), W2)


def _tcc(acc, deg_col, b2, batch3, Wc, bc):
    return pl.pallas_call(
        _tcc_body,
        grid=(N // _BNC,),
        in_specs=[
            pl.BlockSpec((NQ, _BNC, QW), lambda i: (0, i, 0)),
            pl.BlockSpec((_BNC, 1), lambda i: (i, 0)),
            pl.BlockSpec((1, DIM), lambda i: (0, 0)),
            pl.BlockSpec((1, 1, _BNC), lambda i: (i, 0, 0)),
            pl.BlockSpec((DIM, NUM_CLS), lambda i: (0, 0)),
            pl.BlockSpec((1, NUM_CLS), lambda i: (0, 0)),
        ],
        out_specs=pl.BlockSpec((NUM_GRAPHS, NUM_CLS), lambda i: (0, 0)),
        out_shape=jax.ShapeDtypeStruct((NUM_GRAPHS, NUM_CLS), jnp.float32),
        scratch_shapes=[
            pltpu.VMEM((NUM_GRAPHS, DIM), jnp.float32),
            pltpu.VMEM((NUM_GRAPHS, 128), jnp.float32),
        ],
    )(acc, deg_col, b2.reshape(1, DIM), batch3, Wc, bc.reshape(1, NUM_CLS))


def kernel(x, edge_index, batch, W0, b0, W1, b1, W2, b2, Wc, bc):
    src = edge_index[0]
    dst = edge_index[1]
    pad = E_PAD - E
    srcp = jnp.concatenate([src, jnp.full((pad,), N, jnp.int32)])
    dstp = jnp.concatenate([dst, jnp.full((pad,), N, jnp.int32)])
    rs = srcp.reshape(16, CHUNKS_PER_TILE, CHUNK)
    gsrc = jnp.stack([rs + q * N_ACC for q in range(NQ)])
    gdst = dstp.reshape(16, CHUNKS_PER_TILE, CHUNK)
    x_pad = jnp.pad(x, ((0, N_ACC - N), (0, 0)))
    batch3 = batch.reshape(N // _BNC, 1, _BNC)
    ones_in = jnp.ones((CHUNK, 16), jnp.float32)
    zeros_in = jnp.zeros((ROWS_PER_TILE, 16), jnp.float32)

    deg16 = _deg_kernel(gdst, ones_in, zeros_in)
    deg_col = deg16[:, 0:1]

    p1 = _tca(x_pad, W0, b0, W1, deg_col)
    acc1 = _agg_kernel(p1.reshape(NQ * N_ACC, QW), gsrc, gdst)
    p2 = _tcb(acc1.reshape(NQ, N_ACC, QW), deg_col, b1, W2)
    acc2 = _agg_kernel(p2.reshape(NQ * N_ACC, QW), gsrc, gdst)
    return _tcc(acc2.reshape(NQ, N_ACC, QW), deg_col, b2, batch3, Wc, bc)


# speedup vs baseline: 8.5299x; 1.0722x over previous
import functools

import jax
import jax.numpy as jnp
from jax import lax
from jax.experimental import pallas as pl
from jax.experimental.pallas import tpu as pltpu
from jax.experimental.pallas import tpu_sc as plsc

N = 10000
E = 160000
DIM = 256
QW = 64
NQ = 4
NUM_CLS = 64
NUM_GRAPHS = 64

N_ACC = 10240
ROWS_PER_TILE = N_ACC // 16
CHUNK = 128
CHUNKS_PER_TILE = 80
E_PAD = 16 * CHUNKS_PER_TILE * CHUNK
NBUF = 8
GROUPS = CHUNKS_PER_TILE // NBUF

_sc_mesh = plsc.VectorSubcoreMesh(core_axis_name="c", subcore_axis_name="s")
_sc_params = pltpu.CompilerParams(use_tc_tiling_on_sc=False)


def _deg_body(gdst_hbm, ones_hbm, zeros_hbm, deg_out, didx, ones_v, zbuf,
              acc_sh):
    cid = lax.axis_index("c")
    sid = lax.axis_index("s")

    @pl.when(cid == 0)
    def _():
        pltpu.sync_copy(zeros_hbm, zbuf)
        pltpu.sync_copy(zbuf, acc_sh.at[pl.ds(sid * ROWS_PER_TILE,
                                              ROWS_PER_TILE)])
        pltpu.sync_copy(ones_hbm, ones_v)
        pltpu.sync_copy(gdst_hbm.at[sid], didx)

    plsc.subcore_barrier()

    @pl.when(cid == 0)
    def _():
        def body(j, carry):
            pltpu.sync_copy(ones_v, acc_sh.at[didx.at[j]], add=True)
            return carry
        lax.fori_loop(0, CHUNKS_PER_TILE, body, 0)

    plsc.subcore_barrier()

    @pl.when(cid == 0)
    def _():
        pltpu.sync_copy(acc_sh.at[pl.ds(sid * ROWS_PER_TILE, ROWS_PER_TILE)],
                        zbuf)
        pltpu.sync_copy(zbuf, deg_out.at[pl.ds(sid * ROWS_PER_TILE,
                                               ROWS_PER_TILE)])


_deg_kernel = functools.partial(
    pl.kernel,
    out_type=jax.ShapeDtypeStruct((N_ACC, 16), jnp.float32),
    mesh=_sc_mesh,
    scratch_types=[
        pltpu.VMEM((CHUNKS_PER_TILE, CHUNK), jnp.int32),
        pltpu.VMEM((CHUNK, 16), jnp.float32),
        pltpu.VMEM((ROWS_PER_TILE, 16), jnp.float32),
        pltpu.VMEM_SHARED((N_ACC, 16), jnp.float32),
    ],
    compiler_params=_sc_params,
)(_deg_body)


def _agg_body(p_hbm, gsrc_hbm, gdst_hbm, out_hbm, sidx, didx,
              b0, b1, b2, b3, b4, b5, b6, b7,
              s0, s1, s2, s3, s4, s5, s6, s7,
              t0, t1, t2, t3, t4, t5, t6, t7, acc_sh):
    cid = lax.axis_index("c")
    sid = lax.axis_index("s")
    bufs = [b0, b1, b2, b3, b4, b5, b6, b7]
    sems = [s0, s1, s2, s3, s4, s5, s6, s7]
    ssems = [t0, t1, t2, t3, t4, t5, t6, t7]
    rowbase = sid * ROWS_PER_TILE

    pltpu.sync_copy(gdst_hbm.at[sid], didx)

    for qi in range(2):
        q = 2 * cid + qi
        hbase = q * N_ACC + rowbase
        for k in range(ROWS_PER_TILE // CHUNK):
            pltpu.sync_copy(p_hbm.at[pl.ds(hbase + k * CHUNK, CHUNK)], b0)
            pltpu.sync_copy(b0, acc_sh.at[pl.ds(rowbase + k * CHUNK, CHUNK)])
        pltpu.sync_copy(gsrc_hbm.at[q, sid], sidx)

        plsc.subcore_barrier()

        nb = NBUF // 2

        def fire(bank, j0):
            o = bank * nb
            for k in range(nb):
                pltpu.async_copy(p_hbm.at[sidx.at[j0 + k]], bufs[o + k],
                                 sems[o + k])

        def drain_scatter(bank, j0):
            o = bank * nb
            sdescs = []
            for k in range(nb):
                pltpu.make_async_copy(p_hbm.at[sidx.at[j0 + k]], bufs[o + k],
                                      sems[o + k]).wait()
                sdescs.append(
                    pltpu.async_copy(bufs[o + k],
                                     acc_sh.at[didx.at[j0 + k]],
                                     ssems[o + k], add=True))
            for d in sdescs:
                d.wait()

        fire(0, 0)

        def group(g, carry):
            jA = g * NBUF
            jB = jA + nb
            fire(1, jB)
            drain_scatter(0, jA)

            @pl.when(g < GROUPS - 1)
            def _():
                fire(0, jA + NBUF)

            drain_scatter(1, jB)
            return carry

        lax.fori_loop(0, GROUPS, group, 0)

        plsc.subcore_barrier()

        for k in range(ROWS_PER_TILE // CHUNK):
            pltpu.sync_copy(acc_sh.at[pl.ds(rowbase + k * CHUNK, CHUNK)], b0)
            pltpu.sync_copy(b0, out_hbm.at[pl.ds(hbase + k * CHUNK, CHUNK)])


_agg_kernel = functools.partial(
    pl.kernel,
    out_type=jax.ShapeDtypeStruct((NQ * N_ACC, QW), jnp.float32),
    mesh=_sc_mesh,
    scratch_types=[
        pltpu.VMEM((CHUNKS_PER_TILE, CHUNK), jnp.int32),
        pltpu.VMEM((CHUNKS_PER_TILE, CHUNK), jnp.int32),
    ] + [pltpu.VMEM((CHUNK, QW), jnp.float32)] * NBUF
      + [pltpu.SemaphoreType.DMA] * (2 * NBUF)
      + [pltpu.VMEM_SHARED((N_ACC, QW), jnp.float32)],
    compiler_params=_sc_params,
)(_agg_body)


_BN = 1280


def _tca_body(x_ref, w0_ref, b0_ref, w1_ref, deg_ref, out_ref):
    i = pl.program_id(0)
    h = jnp.dot(x_ref[...], w0_ref[...],
                preferred_element_type=jnp.float32) + b0_ref[...]
    t = jnp.dot(h, w1_ref[...], preferred_element_type=jnp.float32)
    dinv = lax.rsqrt(deg_ref[...] + 1.0)
    p = t * dinv
    rows = lax.broadcasted_iota(jnp.int32, (_BN, 1), 0) + i * _BN
    p = jnp.where(rows < N, p, 0.0)
    for q in range(NQ):
        out_ref[q] = p[:, q * QW:(q + 1) * QW]


def _tcb_body(acc_ref, deg_ref, b1_ref, w2_ref, out_ref):
    i = pl.program_id(0)
    a = jnp.concatenate([acc_ref[q] for q in range(NQ)], axis=1)
    dinv = lax.rsqrt(deg_ref[...] + 1.0)
    h1 = jnp.maximum(a * dinv + b1_ref[...], 0.0)
    t = jnp.dot(h1, w2_ref[...], preferred_element_type=jnp.float32)
    p = t * dinv
    rows = lax.broadcasted_iota(jnp.int32, (_BN, 1), 0) + i * _BN
    p = jnp.where(rows < N, p, 0.0)
    for q in range(NQ):
        out_ref[q] = p[:, q * QW:(q + 1) * QW]


_BNC = 1000


def _tcc_body(acc_ref, deg_ref, b2_ref, batch_ref, wc_ref, bc_ref, out_ref,
              sums_ref, cnts_ref):
    i = pl.program_id(0)

    @pl.when(i == 0)
    def _():
        sums_ref[...] = jnp.zeros_like(sums_ref)
        cnts_ref[...] = jnp.zeros_like(cnts_ref)

    a = jnp.concatenate([acc_ref[q] for q in range(NQ)], axis=1)
    dinv = lax.rsqrt(deg_ref[...] + 1.0)
    h2 = a * dinv + b2_ref[...]
    b = batch_ref[0, 0]
    gids = lax.broadcasted_iota(jnp.int32, (NUM_GRAPHS, _BNC), 0)
    bt = lax.broadcast_in_dim(b, (NUM_GRAPHS, _BNC), (1,))
    onehot = (bt == gids).astype(jnp.float32)
    sums_ref[...] += lax.dot_general(
        onehot, h2, (((1,), (0,)), ((), ())),
        preferred_element_type=jnp.float32)
    cnts_ref[...] += jnp.sum(onehot, axis=1, keepdims=True) * jnp.ones(
        (NUM_GRAPHS, 128), jnp.float32)

    @pl.when(i == N // _BNC - 1)
    def _():
        pooled = sums_ref[...] / jnp.maximum(cnts_ref[:, 0:1], 1.0)
        logits = jnp.dot(pooled, wc_ref[...],
                         preferred_element_type=jnp.float32) + bc_ref[...]
        m = jnp.max(logits, axis=1, keepdims=True)
        e = jnp.exp(logits - m)
        out_ref[...] = e / jnp.sum(e, axis=1, keepdims=True)


def _tca(x_pad, W0, b0, W1, deg_col):
    return pl.pallas_call(
        _tca_body,
        grid=(N_ACC // _BN,),
        in_specs=[
            pl.BlockSpec((_BN, DIM), lambda i: (i, 0)),
            pl.BlockSpec((DIM, DIM), lambda i: (0, 0)),
            pl.BlockSpec((1, DIM), lambda i: (0, 0)),
            pl.BlockSpec((DIM, DIM), lambda i: (0, 0)),
            pl.BlockSpec((_BN, 1), lambda i: (i, 0)),
        ],
        out_specs=pl.BlockSpec((NQ, _BN, QW), lambda i: (0, i, 0)),
        out_shape=jax.ShapeDtypeStruct((NQ, N_ACC, QW), jnp.float32),
    )(x_pad, W0, b0.reshape(1, DIM), W1, deg_col)


def _tcb(acc, deg_col, b1, W2):
    return pl.pallas_call(
        _tcb_body,
        grid=(N_ACC // _BN,),
        in_specs=[
            pl.BlockSpec((NQ, _BN, QW), lambda i: (0, i, 0)),
            pl.BlockSpec((_BN, 1), lambda i: (i, 0)),
            pl.BlockSpec((1, DIM), lambda i: (0, 0)),
            pl.BlockSpec((DIM, DIM), lambda i: (0, 0)),
        ],
        out_specs=pl.BlockSpec((NQ, _BN, QW), lambda i: (0, i, 0)),
        out_shape=jax.ShapeDtypeStruct((NQ, N_ACC, QW), jnp.float32),
    )(acc, deg_col, b1.reshape(1, DIM), W2)


def _tcc(acc, deg_col, b2, batch3, Wc, bc):
    return pl.pallas_call(
        _tcc_body,
        grid=(N // _BNC,),
        in_specs=[
            pl.BlockSpec((NQ, _BNC, QW), lambda i: (0, i, 0)),
            pl.BlockSpec((_BNC, 1), lambda i: (i, 0)),
            pl.BlockSpec((1, DIM), lambda i: (0, 0)),
            pl.BlockSpec((1, 1, _BNC), lambda i: (i, 0, 0)),
            pl.BlockSpec((DIM, NUM_CLS), lambda i: (0, 0)),
            pl.BlockSpec((1, NUM_CLS), lambda i: (0, 0)),
        ],
        out_specs=pl.BlockSpec((NUM_GRAPHS, NUM_CLS), lambda i: (0, 0)),
        out_shape=jax.ShapeDtypeStruct((NUM_GRAPHS, NUM_CLS), jnp.float32),
        scratch_shapes=[
            pltpu.VMEM((NUM_GRAPHS, DIM), jnp.float32),
            pltpu.VMEM((NUM_GRAPHS, 128), jnp.float32),
        ],
    )(acc, deg_col, b2.reshape(1, DIM), batch3, Wc, bc.reshape(1, NUM_CLS))


def kernel(x, edge_index, batch, W0, b0, W1, b1, W2, b2, Wc, bc):
    src = edge_index[0]
    dst = edge_index[1]
    pad = E_PAD - E
    srcp = jnp.concatenate([src, jnp.full((pad,), N, jnp.int32)])
    dstp = jnp.concatenate([dst, jnp.full((pad,), N, jnp.int32)])
    rs = srcp.reshape(16, CHUNKS_PER_TILE, CHUNK)
    gsrc = jnp.stack([rs + q * N_ACC for q in range(NQ)])
    gdst = dstp.reshape(16, CHUNKS_PER_TILE, CHUNK)
    x_pad = jnp.pad(x, ((0, N_ACC - N), (0, 0)))
    batch3 = batch.reshape(N // _BNC, 1, _BNC)
    ones_in = jnp.ones((CHUNK, 16), jnp.float32)
    zeros_in = jnp.zeros((ROWS_PER_TILE, 16), jnp.float32)

    deg16 = _deg_kernel(gdst, ones_in, zeros_in)
    deg_col = deg16[:, 0:1]

    p1 = _tca(x_pad, W0, b0, W1, deg_col)
    acc1 = _agg_kernel(p1.reshape(NQ * N_ACC, QW), gsrc, gdst)
    p2 = _tcb(acc1.reshape(NQ, N_ACC, QW), deg_col, b1, W2)
    acc2 = _agg_kernel(p2.reshape(NQ * N_ACC, QW), gsrc, gdst)
    return _tcc(acc2.reshape(NQ, N_ACC, QW), deg_col, b2, batch3, Wc, bc)
